# baseline (speedup 1.0000x reference)
import jax
import jax.numpy as jnp
from jax.experimental import pallas as pl
from jax.experimental.pallas import tpu as pltpu

NUM_EXPERTS = 8
TOP_K = 2
HIDDEN = 1024
BT = 4096


def _router_block(x_ref, w_ref, logits_ref, aff_ref, idx_ref):
    x = x_ref[...]
    w = w_ref[...]
    logits = jax.lax.dot_general(
        w, x,
        dimension_numbers=(((1,), (1,)), ((), ())),
        preferred_element_type=jnp.float32,
    )
    m = jnp.max(logits, axis=0, keepdims=True)
    e = jnp.exp(logits - m)
    s = jnp.sum(e, axis=0, keepdims=True)
    aff = e * (1.0 / s)

    iota = jax.lax.broadcasted_iota(jnp.int32, aff.shape, 0)
    big = jnp.int32(NUM_EXPERTS)
    v1 = jnp.max(aff, axis=0, keepdims=True)
    idx1 = jnp.min(jnp.where(aff == v1, iota, big), axis=0, keepdims=True)
    aff2 = jnp.where(iota == idx1, -1.0, aff)
    v2 = jnp.max(aff2, axis=0, keepdims=True)
    idx2 = jnp.min(jnp.where(aff2 == v2, iota, big), axis=0, keepdims=True)

    logits_ref[...] = logits
    aff_ref[...] = aff
    idx_ref[...] = jnp.concatenate([idx1, idx2], axis=0)


@jax.jit
def _router(x, W):
    T = x.shape[0]
    nblk = T // BT
    logits_w, aff_w, idx_w = pl.pallas_call(
        _router_block,
        grid=(nblk,),
        in_specs=[
            pl.BlockSpec((BT, HIDDEN), lambda i: (i, 0)),
            pl.BlockSpec((NUM_EXPERTS, HIDDEN), lambda i: (0, 0)),
        ],
        out_specs=[
            pl.BlockSpec((NUM_EXPERTS, BT), lambda i: (0, i)),
            pl.BlockSpec((NUM_EXPERTS, BT), lambda i: (0, i)),
            pl.BlockSpec((TOP_K, BT), lambda i: (0, i)),
        ],
        out_shape=[
            jax.ShapeDtypeStruct((NUM_EXPERTS, T), jnp.float32),
            jax.ShapeDtypeStruct((NUM_EXPERTS, T), jnp.float32),
            jax.ShapeDtypeStruct((TOP_K, T), jnp.int32),
        ],
        compiler_params=pltpu.CompilerParams(
            vmem_limit_bytes=50 * 1024 * 1024),
    )(x, W)
    return logits_w.T, aff_w.T, idx_w.T


def kernel(hidden_states, W):
    B, S, H = hidden_states.shape
    x = hidden_states.reshape(B * S, H)
    return _router(x, W)
